# baseline (speedup 1.0000x reference)
import functools

import jax
import jax.numpy as jnp
from jax import lax
from jax.experimental import pallas as pl
from jax.experimental.pallas import tpu as pltpu
from jax.experimental.pallas import tpu_sc as plsc

_NCORES = 2
_NSUB = 16
_NW = _NCORES * _NSUB
_CH = 128
_W = 8
_SUB = 3


def _sc_mesh():
    return plsc.VectorSubcoreMesh(core_axis_name="c", subcore_axis_name="s")


def _chunk_base(c, s, kcha, kchb):
    return jnp.where(c == 0, s * kcha, _NSUB * kcha + s * kchb)


def _make_deg_kernel(npad, kcha, kchb, rpt):
    @functools.partial(
        pl.kernel,
        out_type=jax.ShapeDtypeStruct((_NCORES * npad, _W), jnp.float32),
        mesh=_sc_mesh(),
        scratch_types=[
            pltpu.VMEM((kcha, _CH), jnp.int32),
            pltpu.VMEM((_CH, _W), jnp.float32),
            pltpu.VMEM_SHARED((npad, _W), jnp.float32),
            pltpu.SemaphoreType.DMA,
        ],
        compiler_params=pltpu.CompilerParams(use_tc_tiling_on_sc=False),
    )
    def deg_kernel(dst_hbm, ones_hbm, zeros_hbm, out_hbm, idx_v, ones_v, acc_sh, sem):
        c = lax.axis_index("c")
        s = lax.axis_index("s")
        base = _chunk_base(c, s, kcha, kchb)
        pltpu.sync_copy(dst_hbm.at[pl.ds(base, kcha)], idx_v)
        pltpu.sync_copy(ones_hbm, ones_v)
        pltpu.sync_copy(zeros_hbm, acc_sh.at[pl.ds(s * rpt, rpt)])
        plsc.subcore_barrier()

        ngrp = jnp.where(c == 0, kcha // 8, kchb // 8)

        def grp(g, carry):
            jb = g * 8
            descs = []
            for b in range(8):
                d = pltpu.make_async_copy(ones_v, acc_sh.at[idx_v.at[jb + b]], sem)
                d.start(add=True)
                descs.append(d)
            for d in descs:
                d.wait()
            return carry

        lax.fori_loop(0, ngrp, grp, 0)
        plsc.subcore_barrier()
        pltpu.sync_copy(
            acc_sh.at[pl.ds(s * rpt, rpt)],
            out_hbm.at[pl.ds(c * npad + s * rpt, rpt)],
        )

    return deg_kernel


def _make_agg_kernel(npad, kcha, kchb, rpt):
    @functools.partial(
        pl.kernel,
        out_type=jax.ShapeDtypeStruct((_NCORES * npad, _W), jnp.float32),
        mesh=_sc_mesh(),
        scratch_types=[
            pltpu.VMEM((kcha, _CH), jnp.int32),
            pltpu.VMEM((kcha, _CH), jnp.int32),
            pltpu.VMEM((4 * _SUB, _CH, _W), jnp.float32),
            pltpu.VMEM_SHARED((npad, _W), jnp.float32),
        ] + [pltpu.SemaphoreType.DMA] * 8,
        compiler_params=pltpu.CompilerParams(use_tc_tiling_on_sc=False),
    )
    def agg_kernel(table_hbm, src_hbm, dst_hbm, zeros_hbm, out_hbm,
                   src_v, dst_v, bufs, acc_sh, *sems):
        gs, ss = sems[:4], sems[4:]
        c = lax.axis_index("c")
        s = lax.axis_index("s")
        base0 = _chunk_base(c, s, kcha, kchb)
        pltpu.sync_copy(src_hbm.at[pl.ds(base0, kcha)], src_v)
        pltpu.sync_copy(dst_hbm.at[pl.ds(base0, kcha)], dst_v)
        pltpu.sync_copy(zeros_hbm, acc_sh.at[pl.ds(s * rpt, rpt)])
        plsc.subcore_barrier()

        nsub = jnp.where(c == 0, kcha // _SUB, kchb // _SUB)
        ngrp = jnp.where(c == 0, kcha // (4 * _SUB), kchb // (4 * _SUB))

        def gathers(t, q):
            return [pltpu.make_async_copy(
                table_hbm.at[src_v.at[t * _SUB + b]],
                bufs.at[q * _SUB + b], gs[q]) for b in range(_SUB)]

        def scatters(t, q):
            return [pltpu.make_async_copy(
                bufs.at[q * _SUB + b],
                acc_sh.at[dst_v.at[t * _SUB + b]], ss[q]) for b in range(_SUB)]

        for d in gathers(0, 0) + gathers(1, 1):
            d.start()

        def grp(gi, carry):
            for q in range(4):
                t = 4 * gi + q
                for d in gathers(t, q):
                    d.wait()
                for d in scatters(t, q):
                    d.start(add=True)
                if q < 2:
                    @pl.when(gi > 0)
                    def _():
                        for d in scatters(t - 2, (q + 2) % 4):
                            d.wait()
                else:
                    for d in scatters(t - 2, (q + 2) % 4):
                        d.wait()

                @pl.when(t + 2 < nsub)
                def _():
                    for d in gathers(t + 2, (q + 2) % 4):
                        d.start()

            return carry

        lax.fori_loop(0, ngrp, grp, 0)
        for d in scatters(nsub - 2, 2) + scatters(nsub - 1, 3):
            d.wait()
        plsc.subcore_barrier()
        pltpu.sync_copy(
            acc_sh.at[pl.ds(s * rpt, rpt)],
            out_hbm.at[pl.ds(c * npad + s * rpt, rpt)],
        )

    return agg_kernel


def _make_prep_body(g):
    def _prep_body(xr_ref, k1_ref, degpk_ref, h1p_ref, dinv_ref):
        deg = degpk_ref[0:g, :] + degpk_ref[g:2 * g, :] + 1.0
        dinv = lax.rsqrt(deg)
        h1 = jnp.dot(xr_ref[...], k1_ref[...], preferred_element_type=jnp.float32)
        h1p_ref[...] = h1 * dinv
        dinv_ref[...] = dinv

    return _prep_body


def _make_mid_body(g):
    def _mid_body(pts_ref, h1p_ref, dinv_ref, b1_ref, k2_ref, h2p_ref):
        dinv = dinv_ref[...]
        z = dinv * (pts_ref[0:g, :] + pts_ref[g:2 * g, :] + h1p_ref[...]) + b1_ref[...]
        o1 = jnp.maximum(z, 0.0)
        h2 = jnp.dot(o1, k2_ref[...], preferred_element_type=jnp.float32)
        h2p_ref[...] = h2 * dinv

    return _mid_body


def _make_fin_body(g, ncls):
    def _fin_body(pts_ref, h2p_ref, dinv_ref, b2_ref, out_ref):
        z = dinv_ref[...] * (pts_ref[0:g, :] + pts_ref[g:2 * g, :] + h2p_ref[...]) \
            + b2_ref[...]
        lane = lax.broadcasted_iota(jnp.int32, z.shape, 1) % _W
        t = jnp.where(lane < ncls, z, -1e30)
        m0 = t
        for sh in range(1, ncls):
            m0 = jnp.maximum(m0, jnp.roll(t, -sh, axis=1))
        mm = jnp.where(lane == 0, m0, 0.0)
        a = lax.broadcasted_iota(jnp.int32, (16 * _W, 16 * _W), 0)
        b = lax.broadcasted_iota(jnp.int32, (16 * _W, 16 * _W), 1)
        same_grp = (a // _W) == (b // _W)
        g0 = jnp.where(same_grp & ((a % _W) == 0), 1.0, 0.0)
        gb = jnp.where(same_grp, 1.0, 0.0)
        mb = jnp.dot(mm, g0, preferred_element_type=jnp.float32)
        e = jnp.where(lane < ncls, jnp.exp(z - mb), 0.0)
        s = jnp.dot(e, gb, preferred_element_type=jnp.float32)
        out_ref[...] = z - mb - jnp.log(s)

    return _fin_body


def kernel(x, edge_index, W1, b1, W2, b2):
    n, d_in = x.shape
    e = edge_index.shape[1]
    h = W1.shape[1]
    ncls = W2.shape[1]
    assert h <= _W and ncls <= _W

    npad = -(-n // (_NSUB * 16)) * (_NSUB * 16)
    if npad == n:
        npad += _NSUB * 16
    rpt = npad // _NSUB
    ksum = -(-(-(-e // _CH)) // _NSUB)
    q4 = 4 * _SUB
    kcha = max(q4, -(-(ksum * 6 // 10) // q4) * q4)
    kchb = max(q4, -(-(ksum - kcha) // q4) * q4)
    tch = _NSUB * (kcha + kchb)
    tchp = tch + (kcha - kchb)
    epad = tch * _CH

    g = npad // 16
    lanes = 16 * _W

    if e % _CH == 0:
        tc0 = e // _CH
        ei3 = edge_index.reshape(2, tc0, _CH).transpose(1, 0, 2)
        pad = jnp.full((tchp - tc0, _CH), n, dtype=jnp.int32)
        srcw = jnp.concatenate([ei3[:, 0, :], pad], axis=0)
        dstw = jnp.concatenate([ei3[:, 1, :], pad], axis=0)
    else:
        src = jnp.full((tchp * _CH,), n, dtype=jnp.int32).at[:e].set(edge_index[0])
        dst = jnp.full((tchp * _CH,), n, dtype=jnp.int32).at[:e].set(edge_index[1])
        srcw = src.reshape(tchp, _CH)
        dstw = dst.reshape(tchp, _CH)
    xpad = jnp.zeros((npad, d_in), x.dtype).at[:n].set(x)
    xr = xpad.reshape(g, 16 * d_in)
    w2p = jnp.zeros((_W, _W), W2.dtype).at[:h, :ncls].set(W2)
    eye16 = jnp.eye(16, dtype=jnp.float32)
    k1 = jnp.kron(eye16, W1.astype(jnp.float32))
    k2 = jnp.kron(eye16, w2p.astype(jnp.float32))
    b1t = jnp.tile(jnp.zeros((_W,), jnp.float32).at[:h].set(b1), 16)[None, :]
    b2t = jnp.tile(jnp.zeros((_W,), jnp.float32).at[:ncls].set(b2), 16)[None, :]
    ones = jnp.ones((_CH, _W), jnp.float32)
    zeros = jnp.zeros((rpt, _W), jnp.float32)

    deg_sc = _make_deg_kernel(npad, kcha, kchb, rpt)
    agg_sc = _make_agg_kernel(npad, kcha, kchb, rpt)

    full = lambda shape: pl.BlockSpec(shape, lambda: (0,) * len(shape))

    degpk = deg_sc(dstw, ones, zeros).reshape(2 * g, lanes)

    h1pk, dinvk = pl.pallas_call(
        _make_prep_body(g),
        in_specs=[full((g, 16 * d_in)), full((16 * d_in, lanes)),
                  full((2 * g, lanes))],
        out_specs=[full((g, lanes)), full((g, lanes))],
        out_shape=[
            jax.ShapeDtypeStruct((g, lanes), jnp.float32),
            jax.ShapeDtypeStruct((g, lanes), jnp.float32),
        ],
    )(xr, k1, degpk)

    parts1 = agg_sc(h1pk.reshape(npad, _W), srcw, dstw, zeros)

    h2pk = pl.pallas_call(
        _make_mid_body(g),
        in_specs=[full((2 * g, lanes)), full((g, lanes)), full((g, lanes)),
                  full((1, lanes)), full((lanes, lanes))],
        out_specs=full((g, lanes)),
        out_shape=jax.ShapeDtypeStruct((g, lanes), jnp.float32),
    )(parts1.reshape(2 * g, lanes), h1pk, dinvk, b1t, k2)

    parts2 = agg_sc(h2pk.reshape(npad, _W), srcw, dstw, zeros)

    outpk = pl.pallas_call(
        _make_fin_body(g, ncls),
        in_specs=[full((2 * g, lanes)), full((g, lanes)), full((g, lanes)),
                  full((1, lanes))],
        out_specs=full((g, lanes)),
        out_shape=jax.ShapeDtypeStruct((g, lanes), jnp.float32),
    )(parts2.reshape(2 * g, lanes), h2pk, dinvk, b2t)

    return outpk.reshape(npad, _W)[:n, :ncls]


# speedup vs baseline: 1.9728x; 1.9728x over previous
import functools

import jax
import jax.numpy as jnp
from jax import lax
from jax.experimental import pallas as pl
from jax.experimental.pallas import tpu as pltpu
from jax.experimental.pallas import tpu_sc as plsc

_NCORES = 2
_NSUB = 16
_NW = _NCORES * _NSUB
_CH = 128
_W = 8
_NBUF = 16
_LEAD = 8


def _sc_mesh():
    return plsc.VectorSubcoreMesh(core_axis_name="c", subcore_axis_name="s")


def _chunk_base(c, s, kcha, kchb):
    return jnp.where(c == 0, s * kcha, _NSUB * kcha + s * kchb)


def _make_deg_kernel(npad, kcha, kchb, rpt):
    @functools.partial(
        pl.kernel,
        out_type=jax.ShapeDtypeStruct((_NCORES * npad, _W), jnp.float32),
        mesh=_sc_mesh(),
        scratch_types=[
            pltpu.VMEM((kcha, _CH), jnp.int32),
            pltpu.VMEM((_CH, _W), jnp.float32),
            pltpu.VMEM_SHARED((npad, _W), jnp.float32),
            pltpu.SemaphoreType.DMA,
        ],
        compiler_params=pltpu.CompilerParams(use_tc_tiling_on_sc=False),
    )
    def deg_kernel(dst_hbm, ones_hbm, zeros_hbm, out_hbm, idx_v, ones_v, acc_sh, sem):
        c = lax.axis_index("c")
        s = lax.axis_index("s")
        base = _chunk_base(c, s, kcha, kchb)
        pltpu.sync_copy(dst_hbm.at[pl.ds(base, kcha)], idx_v)
        pltpu.sync_copy(ones_hbm, ones_v)
        pltpu.sync_copy(zeros_hbm, acc_sh.at[pl.ds(s * rpt, rpt)])
        plsc.subcore_barrier()

        ngrp = jnp.where(c == 0, kcha // 8, kchb // 8)

        def grp(g, carry):
            jb = g * 8
            descs = []
            for b in range(8):
                d = pltpu.make_async_copy(ones_v, acc_sh.at[idx_v.at[jb + b]], sem)
                d.start(add=True)
                descs.append(d)
            for d in descs:
                d.wait()
            return carry

        lax.fori_loop(0, ngrp, grp, 0)
        plsc.subcore_barrier()
        pltpu.sync_copy(
            acc_sh.at[pl.ds(s * rpt, rpt)],
            out_hbm.at[pl.ds(c * npad + s * rpt, rpt)],
        )

    return deg_kernel


def _make_agg_kernel(npad, kcha, kchb, rpt):
    @functools.partial(
        pl.kernel,
        out_type=jax.ShapeDtypeStruct((_NCORES * npad, _W), jnp.float32),
        mesh=_sc_mesh(),
        scratch_types=[
            pltpu.VMEM((kcha, _CH), jnp.int32),
            pltpu.VMEM((kcha, _CH), jnp.int32),
            pltpu.VMEM((16, _CH, _W), jnp.float32),
            pltpu.VMEM_SHARED((npad, _W), jnp.float32),
            pltpu.SemaphoreType.DMA,
            pltpu.SemaphoreType.DMA,
            pltpu.SemaphoreType.DMA,
            pltpu.SemaphoreType.DMA,
        ],
        compiler_params=pltpu.CompilerParams(use_tc_tiling_on_sc=False),
    )
    def agg_kernel(table_hbm, src_hbm, dst_hbm, zeros_hbm, out_hbm,
                   src_v, dst_v, bufs, acc_sh, gsa, gsb, ssa, ssb):
        c = lax.axis_index("c")
        s = lax.axis_index("s")
        base0 = _chunk_base(c, s, kcha, kchb)
        pltpu.sync_copy(src_hbm.at[pl.ds(base0, kcha)], src_v)
        pltpu.sync_copy(dst_hbm.at[pl.ds(base0, kcha)], dst_v)
        pltpu.sync_copy(zeros_hbm, acc_sh.at[pl.ds(s * rpt, rpt)])
        plsc.subcore_barrier()

        ngrp = jnp.where(c == 0, kcha // 16, kchb // 16)

        def gathers(jb, lo, sem):
            return [pltpu.make_async_copy(
                table_hbm.at[src_v.at[jb + lo + b]], bufs.at[lo + b], sem)
                for b in range(8)]

        def scatters(jb, lo, sem):
            return [pltpu.make_async_copy(
                bufs.at[lo + b], acc_sh.at[dst_v.at[jb + lo + b]], sem)
                for b in range(8)]

        for d in gathers(0, 0, gsa):
            d.start()

        def grp(gi, carry):
            jb = gi * 16
            for d in gathers(jb, 0, gsa):
                d.wait()

            @pl.when(gi > 0)
            def _():
                for d in scatters(jb - 16, 8, ssb):
                    d.wait()

            for d in scatters(jb, 0, ssa):
                d.start(add=True)
            for d in gathers(jb, 8, gsb):
                d.start()
            for d in gathers(jb, 8, gsb):
                d.wait()
            for d in scatters(jb, 0, ssa):
                d.wait()
            for d in scatters(jb, 8, ssb):
                d.start(add=True)

            @pl.when(gi + 1 < ngrp)
            def _():
                for d in gathers(jb + 16, 0, gsa):
                    d.start()

            return carry

        lax.fori_loop(0, ngrp, grp, 0)
        for d in scatters((ngrp - 1) * 16, 8, ssb):
            d.wait()
        plsc.subcore_barrier()
        pltpu.sync_copy(
            acc_sh.at[pl.ds(s * rpt, rpt)],
            out_hbm.at[pl.ds(c * npad + s * rpt, rpt)],
        )

    return agg_kernel


def _make_prep_body(g):
    def _prep_body(xr_ref, k1_ref, degpk_ref, h1p_ref, dinv_ref):
        deg = degpk_ref[0:g, :] + degpk_ref[g:2 * g, :] + 1.0
        dinv = lax.rsqrt(deg)
        h1 = jnp.dot(xr_ref[...], k1_ref[...], preferred_element_type=jnp.float32)
        h1p_ref[...] = h1 * dinv
        dinv_ref[...] = dinv

    return _prep_body


def _make_mid_body(g):
    def _mid_body(pts_ref, h1p_ref, dinv_ref, b1_ref, k2_ref, h2p_ref):
        dinv = dinv_ref[...]
        z = dinv * (pts_ref[0:g, :] + pts_ref[g:2 * g, :] + h1p_ref[...]) + b1_ref[...]
        o1 = jnp.maximum(z, 0.0)
        h2 = jnp.dot(o1, k2_ref[...], preferred_element_type=jnp.float32)
        h2p_ref[...] = h2 * dinv

    return _mid_body


def _make_fin_body(g, ncls):
    def _fin_body(pts_ref, h2p_ref, dinv_ref, b2_ref, out_ref):
        z = dinv_ref[...] * (pts_ref[0:g, :] + pts_ref[g:2 * g, :] + h2p_ref[...]) \
            + b2_ref[...]
        lane = lax.broadcasted_iota(jnp.int32, z.shape, 1) % _W
        t = jnp.where(lane < ncls, z, -1e30)
        m0 = t
        for sh in range(1, ncls):
            m0 = jnp.maximum(m0, jnp.roll(t, -sh, axis=1))
        mm = jnp.where(lane == 0, m0, 0.0)
        a = lax.broadcasted_iota(jnp.int32, (16 * _W, 16 * _W), 0)
        b = lax.broadcasted_iota(jnp.int32, (16 * _W, 16 * _W), 1)
        same_grp = (a // _W) == (b // _W)
        g0 = jnp.where(same_grp & ((a % _W) == 0), 1.0, 0.0)
        gb = jnp.where(same_grp, 1.0, 0.0)
        mb = jnp.dot(mm, g0, preferred_element_type=jnp.float32)
        e = jnp.where(lane < ncls, jnp.exp(z - mb), 0.0)
        s = jnp.dot(e, gb, preferred_element_type=jnp.float32)
        out_ref[...] = z - mb - jnp.log(s)

    return _fin_body


def kernel(x, edge_index, W1, b1, W2, b2):
    n, d_in = x.shape
    e = edge_index.shape[1]
    h = W1.shape[1]
    ncls = W2.shape[1]
    assert h <= _W and ncls <= _W

    npad = -(-n // (_NSUB * 16)) * (_NSUB * 16)
    if npad == n:
        npad += _NSUB * 16
    rpt = npad // _NSUB
    ksum = -(-(-(-e // _CH)) // _NSUB)
    kcha = max(16, ((ksum * 6 // 10 + 15) // 16) * 16)
    kchb = max(16, ((ksum - kcha + 15) // 16) * 16)
    tch = _NSUB * (kcha + kchb)
    tchp = tch + (kcha - kchb)
    epad = tch * _CH

    g = npad // 16
    lanes = 16 * _W

    if e % _CH == 0:
        tc0 = e // _CH
        ei3 = edge_index.reshape(2, tc0, _CH).transpose(1, 0, 2)
        pad = jnp.full((tchp - tc0, _CH), n, dtype=jnp.int32)
        srcw = jnp.concatenate([ei3[:, 0, :], pad], axis=0)
        dstw = jnp.concatenate([ei3[:, 1, :], pad], axis=0)
    else:
        src = jnp.full((tchp * _CH,), n, dtype=jnp.int32).at[:e].set(edge_index[0])
        dst = jnp.full((tchp * _CH,), n, dtype=jnp.int32).at[:e].set(edge_index[1])
        srcw = src.reshape(tchp, _CH)
        dstw = dst.reshape(tchp, _CH)
    xpad = jnp.zeros((npad, d_in), x.dtype).at[:n].set(x)
    xr = xpad.reshape(g, 16 * d_in)
    w2p = jnp.zeros((_W, _W), W2.dtype).at[:h, :ncls].set(W2)
    eye16 = jnp.eye(16, dtype=jnp.float32)
    k1 = jnp.kron(eye16, W1.astype(jnp.float32))
    k2 = jnp.kron(eye16, w2p.astype(jnp.float32))
    b1t = jnp.tile(jnp.zeros((_W,), jnp.float32).at[:h].set(b1), 16)[None, :]
    b2t = jnp.tile(jnp.zeros((_W,), jnp.float32).at[:ncls].set(b2), 16)[None, :]
    ones = jnp.ones((_CH, _W), jnp.float32)
    zeros = jnp.zeros((rpt, _W), jnp.float32)

    deg_sc = _make_deg_kernel(npad, kcha, kchb, rpt)
    agg_sc = _make_agg_kernel(npad, kcha, kchb, rpt)

    full = lambda shape: pl.BlockSpec(shape, lambda: (0,) * len(shape))

    degpk = deg_sc(dstw, ones, zeros).reshape(2 * g, lanes)

    h1pk, dinvk = pl.pallas_call(
        _make_prep_body(g),
        in_specs=[full((g, 16 * d_in)), full((16 * d_in, lanes)),
                  full((2 * g, lanes))],
        out_specs=[full((g, lanes)), full((g, lanes))],
        out_shape=[
            jax.ShapeDtypeStruct((g, lanes), jnp.float32),
            jax.ShapeDtypeStruct((g, lanes), jnp.float32),
        ],
    )(xr, k1, degpk)

    parts1 = agg_sc(h1pk.reshape(npad, _W), srcw, dstw, zeros)

    h2pk = pl.pallas_call(
        _make_mid_body(g),
        in_specs=[full((2 * g, lanes)), full((g, lanes)), full((g, lanes)),
                  full((1, lanes)), full((lanes, lanes))],
        out_specs=full((g, lanes)),
        out_shape=jax.ShapeDtypeStruct((g, lanes), jnp.float32),
    )(parts1.reshape(2 * g, lanes), h1pk, dinvk, b1t, k2)

    parts2 = agg_sc(h2pk.reshape(npad, _W), srcw, dstw, zeros)

    outpk = pl.pallas_call(
        _make_fin_body(g, ncls),
        in_specs=[full((2 * g, lanes)), full((g, lanes)), full((g, lanes)),
                  full((1, lanes))],
        out_specs=full((g, lanes)),
        out_shape=jax.ShapeDtypeStruct((g, lanes), jnp.float32),
    )(parts2.reshape(2 * g, lanes), h2pk, dinvk, b2t)

    return outpk.reshape(npad, _W)[:n, :ncls]
